# baseline (speedup 1.0000x reference)
import functools

import jax
import jax.numpy as jnp
from jax import lax
from jax.experimental import pallas as pl
from jax.experimental.pallas import tpu as pltpu
from jax.experimental.pallas import tpu_sc as plsc

EPS_ = 1e-08
B_, C_ = 16384, 1000
CP_ = 1024
NC_, NS_, L_ = 2, 16, 16
NW_ = NC_ * NS_
HSC_ = 4096
RPW_ = HSC_ // NW_
PC_ = 128
NP_ = 8
PCL_ = C_ - (NP_ - 1) * PC_
NSB_ = RPW_ // L_


_GDN = lax.GatherDimensionNumbers(
    offset_dims=(), collapsed_slice_dims=(0,), start_index_map=(0,))


def _perm(v, idx):
    return lax.gather(v, idx[:, None], _GDN, (1,),
                      mode=lax.GatherScatterMode.PROMISE_IN_BOUNDS)


def _transpose16(vs, lane):
    for d in (1, 2, 4, 8):
        nv = list(vs)
        for i in range(L_):
            if i & d:
                continue
            j = i | d
            a, b = vs[i], vs[j]
            sa = _perm(a, lane ^ d)
            sb = _perm(b, lane ^ d)
            nv[i] = jnp.where((lane & d) == 0, a, sb)
            nv[j] = jnp.where((lane & d) == 0, sa, b)
        vs = nv
    return vs


def _sc_body(xt_hbm, y_hbm, w_hbm, out_hbm, idx_v, xb, wb2, sv,
             sx0, sx1, sw0, sw1):
    wid = lax.axis_index("s") * NC_ + lax.axis_index("c")
    col0 = wid * RPW_
    pltpu.sync_copy(y_hbm.at[pl.ds(col0, RPW_)], idx_v)
    sx = (sx0, sx1)
    sw = (sw0, sw1)
    lane = jnp.arange(L_, dtype=jnp.int32)

    def pc(p):
        return PC_ if p < NP_ - 1 else PCL_

    def start(p, b):
        c0 = p * PC_
        pltpu.async_copy(
            xt_hbm.at[pl.ds(c0, pc(p)), pl.ds(col0, RPW_)],
            xb.at[b, pl.ds(0, pc(p))], sx[b])
        pltpu.async_copy(
            w_hbm.at[idx_v, pl.ds(c0, PC_)], wb2.at[b], sw[b])

    def wait(p, b):
        pltpu.make_async_copy(
            xt_hbm.at[pl.ds(0, pc(p)), pl.ds(0, RPW_)],
            xb.at[b, pl.ds(0, pc(p))], sx[b]).wait()
        pltpu.make_async_copy(
            w_hbm.at[idx_v, pl.ds(0, PC_)], wb2.at[b], sw[b]).wait()

    zero = jnp.zeros((L_,), jnp.float32)
    for sb in range(NSB_):
        sv[pl.ds(sb * L_, L_)] = zero

    start(0, 0)
    for p in range(NP_):
        b = p % 2
        if p + 1 < NP_:
            start(p + 1, 1 - b)
        wait(p, b)

        def sb_body(sb, carry, b=b):
            acc0 = sv[pl.ds(sb * L_, L_)]

            def k_body(k, acc, b=b, sb=sb):
                vs = [wb2[b, sb * L_ + r, pl.ds(k * L_, L_)]
                      for r in range(L_)]
                cols = _transpose16(vs, lane)
                for cc in range(L_):
                    acc = acc + (xb[b, k * L_ + cc, pl.ds(sb * L_, L_)]
                                 * cols[cc])
                return acc

            acc = lax.fori_loop(0, PC_ // L_, k_body, acc0)
            sv[pl.ds(sb * L_, L_)] = acc
            return carry

        lax.fori_loop(0, NSB_, sb_body, 0)

    pltpu.sync_copy(sv, out_hbm.at[pl.ds(col0, RPW_)])


_sc_call = pl.kernel(
    _sc_body,
    out_type=jax.ShapeDtypeStruct((HSC_,), jnp.float32),
    mesh=plsc.VectorSubcoreMesh(
        core_axis_name="c", subcore_axis_name="s",
        num_cores=NC_, num_subcores=NS_),
    scratch_types=[
        pltpu.VMEM((RPW_,), jnp.int32),
        pltpu.VMEM((2, PC_, RPW_), jnp.float32),
        pltpu.VMEM((2, RPW_, PC_), jnp.float32),
        pltpu.VMEM((RPW_,), jnp.float32),
        pltpu.SemaphoreType.DMA,
        pltpu.SemaphoreType.DMA,
        pltpu.SemaphoreType.DMA,
        pltpu.SemaphoreType.DMA,
    ],
)

BLK_ = 2048


def _tc_body(y_ref, xt_ref, w_ref, out_ref):
    i = pl.program_id(0)
    y_row = y_ref[0]
    classes = jax.lax.broadcasted_iota(jnp.int32, (C_, BLK_), 0)
    onehot_t = (classes == y_row).astype(jnp.bfloat16)
    w_b = w_ref[...].astype(jnp.bfloat16)
    wy_t = jax.lax.dot_general(
        w_b, onehot_t, (((0,), (0,)), ((), ())),
        preferred_element_type=jnp.float32)
    s = jnp.sum(wy_t * xt_ref[...], axis=0, keepdims=True)
    part = jnp.sum(-jnp.log(s + EPS_)).reshape(1, 1)

    @pl.when(i == 0)
    def _():
        out_ref[...] = jnp.zeros((1, 1), jnp.float32)

    out_ref[...] += part


def _fin_body(s_ref, t_ref, out_ref):
    out_ref[...] = (jnp.sum(-jnp.log(s_ref[...] + EPS_)).reshape(1, 1)
                    + t_ref[...])


@jax.jit
def kernel(x, y, w):
    y32 = y.astype(jnp.int32)
    xt = x.T
    w_pad = jnp.pad(w, ((0, 0), (0, CP_ - C_)))
    s_sc = _sc_call(xt, y32, w_pad)

    nblk = (B_ - HSC_) // BLK_
    off = HSC_ // BLK_
    y3 = y32.reshape(B_ // BLK_, 1, BLK_)
    tc_tot = pl.pallas_call(
        _tc_body,
        grid=(nblk,),
        in_specs=[
            pl.BlockSpec((1, 1, BLK_), lambda i: (i + off, 0, 0)),
            pl.BlockSpec((C_, BLK_), lambda i: (0, i + off)),
            pl.BlockSpec((C_, C_), lambda i: (0, 0)),
        ],
        out_specs=pl.BlockSpec((1, 1), lambda i: (0, 0)),
        out_shape=jax.ShapeDtypeStruct((1, 1), jnp.float32),
    )(y3, xt, w)

    total = pl.pallas_call(
        _fin_body,
        in_specs=[
            pl.BlockSpec((HSC_ // 128, 128), lambda: (0, 0)),
            pl.BlockSpec((1, 1), lambda: (0, 0)),
        ],
        out_specs=pl.BlockSpec((1, 1), lambda: (0, 0)),
        out_shape=jax.ShapeDtypeStruct((1, 1), jnp.float32),
    )(s_sc.reshape(HSC_ // 128, 128), tc_tot)
    return total[0, 0] / B_


# speedup vs baseline: 1.0384x; 1.0384x over previous
import functools

import jax
import jax.numpy as jnp
from jax import lax
from jax.experimental import pallas as pl
from jax.experimental.pallas import tpu as pltpu
from jax.experimental.pallas import tpu_sc as plsc

EPS_ = 1e-08
B_, C_ = 16384, 1000
NC_, NS_, L_ = 2, 16, 16
NW_ = NC_ * NS_
HSC_ = 4096
RPW_ = HSC_ // NW_
PC_ = 128
NP_ = 7
TC0_ = NP_ * PC_
TCN_ = C_ - TC0_
NSB_ = RPW_ // L_


_GDN = lax.GatherDimensionNumbers(
    offset_dims=(), collapsed_slice_dims=(0,), start_index_map=(0,))


def _perm(v, idx):
    return lax.gather(v, idx[:, None], _GDN, (1,),
                      mode=lax.GatherScatterMode.PROMISE_IN_BOUNDS)


def _transpose16(vs, lane):
    for d in (1, 2, 4, 8):
        nv = list(vs)
        for i in range(L_):
            if i & d:
                continue
            j = i | d
            a, b = vs[i], vs[j]
            sa = _perm(a, lane ^ d)
            sb = _perm(b, lane ^ d)
            nv[i] = jnp.where((lane & d) == 0, a, sb)
            nv[j] = jnp.where((lane & d) == 0, sa, b)
        vs = nv
    return vs


def _sc_body(xt_hbm, y_hbm, w_hbm, out_hbm, idx_v, xb, wb2, sv,
             sx0, sx1, sw0, sw1):
    wid = lax.axis_index("s") * NC_ + lax.axis_index("c")
    col0 = wid * RPW_
    pltpu.sync_copy(y_hbm.at[pl.ds(col0, RPW_)], idx_v)
    sx = (sx0, sx1)
    sw = (sw0, sw1)
    lane = jnp.arange(L_, dtype=jnp.int32)

    def start(p, b):
        c0 = p * PC_
        pltpu.async_copy(
            xt_hbm.at[pl.ds(c0, PC_), pl.ds(col0, RPW_)], xb.at[b], sx[b])
        pltpu.async_copy(
            w_hbm.at[idx_v, pl.ds(c0, PC_)], wb2.at[b], sw[b])

    def wait(b):
        pltpu.make_async_copy(
            xt_hbm.at[pl.ds(0, PC_), pl.ds(0, RPW_)], xb.at[b], sx[b]).wait()
        pltpu.make_async_copy(
            w_hbm.at[idx_v, pl.ds(0, PC_)], wb2.at[b], sw[b]).wait()

    zero = jnp.zeros((L_,), jnp.float32)
    for sb in range(NSB_):
        sv[pl.ds(sb * L_, L_)] = zero

    start(0, 0)
    for p in range(NP_):
        b = p % 2
        if p + 1 < NP_:
            start(p + 1, 1 - b)
        wait(b)

        def sb_body(sb, carry, b=b):
            acc0 = sv[pl.ds(sb * L_, L_)]

            def k_body(k, acc, b=b, sb=sb):
                vs = [wb2[b, sb * L_ + r, pl.ds(k * L_, L_)]
                      for r in range(L_)]
                cols = _transpose16(vs, lane)
                for cc in range(L_):
                    acc = acc + (xb[b, k * L_ + cc, pl.ds(sb * L_, L_)]
                                 * cols[cc])
                return acc

            acc = lax.fori_loop(0, PC_ // L_, k_body, acc0)
            sv[pl.ds(sb * L_, L_)] = acc
            return carry

        lax.fori_loop(0, NSB_, sb_body, 0)

    pltpu.sync_copy(sv, out_hbm.at[pl.ds(col0, RPW_)])


_sc_call = pl.kernel(
    _sc_body,
    out_type=jax.ShapeDtypeStruct((HSC_,), jnp.float32),
    mesh=plsc.VectorSubcoreMesh(
        core_axis_name="c", subcore_axis_name="s",
        num_cores=NC_, num_subcores=NS_),
    scratch_types=[
        pltpu.VMEM((RPW_,), jnp.int32),
        pltpu.VMEM((2, PC_, RPW_), jnp.float32),
        pltpu.VMEM((2, RPW_, PC_), jnp.float32),
        pltpu.VMEM((RPW_,), jnp.float32),
        pltpu.SemaphoreType.DMA,
        pltpu.SemaphoreType.DMA,
        pltpu.SemaphoreType.DMA,
        pltpu.SemaphoreType.DMA,
    ],
)

BLK_ = 2048


def _tc_body(y_ref, xt_ref, w_ref, out_ref):
    i = pl.program_id(0)
    y_row = y_ref[0]
    classes = jax.lax.broadcasted_iota(jnp.int32, (C_, BLK_), 0)
    onehot_t = (classes == y_row).astype(jnp.bfloat16)
    w_b = w_ref[...].astype(jnp.bfloat16)
    wy_t = jax.lax.dot_general(
        w_b, onehot_t, (((0,), (0,)), ((), ())),
        preferred_element_type=jnp.float32)
    s = jnp.sum(wy_t * xt_ref[...], axis=0, keepdims=True)
    part = jnp.sum(-jnp.log(s + EPS_)).reshape(1, 1)

    @pl.when(i == 0)
    def _():
        out_ref[...] = jnp.zeros((1, 1), jnp.float32)

    out_ref[...] += part


def _fin_body(s_ref, y_ref, xt_ref, w_ref, t_ref, out_ref):
    y_row = y_ref[...]
    classes = jax.lax.broadcasted_iota(jnp.int32, (C_, HSC_), 0)
    onehot_t = (classes == y_row).astype(jnp.bfloat16)
    w_tail = w_ref[:, TC0_:C_].astype(jnp.bfloat16)
    wy_t = jax.lax.dot_general(
        w_tail, onehot_t, (((0,), (0,)), ((), ())),
        preferred_element_type=jnp.float32)
    s_tail = jnp.sum(wy_t * xt_ref[0:TCN_, :], axis=0, keepdims=True)
    s = s_ref[...] + s_tail
    out_ref[...] = (jnp.sum(-jnp.log(s + EPS_)).reshape(1, 1)
                    + t_ref[...])


@jax.jit
def kernel(x, y, w):
    y32 = y.astype(jnp.int32)
    xt = x.T
    s_sc = _sc_call(xt, y32, w)

    nblk = (B_ - HSC_) // BLK_
    off = HSC_ // BLK_
    y3 = y32.reshape(B_ // BLK_, 1, BLK_)
    tc_tot = pl.pallas_call(
        _tc_body,
        grid=(nblk,),
        in_specs=[
            pl.BlockSpec((1, 1, BLK_), lambda i: (i + off, 0, 0)),
            pl.BlockSpec((C_, BLK_), lambda i: (0, i + off)),
            pl.BlockSpec((C_, C_), lambda i: (0, 0)),
        ],
        out_specs=pl.BlockSpec((1, 1), lambda i: (0, 0)),
        out_shape=jax.ShapeDtypeStruct((1, 1), jnp.float32),
    )(y3, xt, w)

    total = pl.pallas_call(
        _fin_body,
        grid=(1,),
        in_specs=[
            pl.BlockSpec((1, HSC_), lambda i: (0, 0)),
            pl.BlockSpec((1, HSC_), lambda i: (0, 0)),
            pl.BlockSpec((PC_, HSC_), lambda i: (NP_, 0)),
            pl.BlockSpec((C_, C_), lambda i: (0, 0)),
            pl.BlockSpec((1, 1), lambda i: (0, 0)),
        ],
        out_specs=pl.BlockSpec((1, 1), lambda i: (0, 0)),
        out_shape=jax.ShapeDtypeStruct((1, 1), jnp.float32),
    )(s_sc.reshape(1, HSC_), y32.reshape(1, B_), xt, w, tc_tot)
    return total[0, 0] / B_
